# baseline (speedup 1.0000x reference)
import functools

import jax
import jax.numpy as jnp
from jax import lax
from jax.experimental import pallas as pl
from jax.experimental.pallas import tpu as pltpu
from jax.experimental.pallas import tpu_sc as plsc

import math

N = 10000
D = 128
N_PAD = 10240
NC, NS = 2, 16
NW = NC * NS
BATCH = 32
NBUF = 8
ROWS_PER_TILE = N_PAD // NS

_mesh = plsc.VectorSubcoreMesh(core_axis_name="c", subcore_axis_name="s")


def _edge_pad(e):
    q = NW * BATCH * math.lcm(8, NBUF)
    return ((e + q - 1) // q) * q


def _make_deg_kernel(e_pad, w=16):
    nb = e_pad // (NW * BATCH)

    @functools.partial(
        pl.kernel,
        out_type=jax.ShapeDtypeStruct((NC, N_PAD, w), jnp.float32),
        mesh=_mesh,
        scratch_types=[
            pltpu.VMEM((nb, BATCH), jnp.int32),
            pltpu.VMEM((BATCH, w), jnp.float32),
            pltpu.VMEM_SHARED((N_PAD, w), jnp.float32),
        ],
    )
    def deg_kernel(dst_hbm, ones_hbm, zeros_hbm, out_hbm, dst_v, ones_v, acc):
        c = lax.axis_index("c")
        s = lax.axis_index("s")
        wid = c * NS + s
        pltpu.sync_copy(dst_hbm.at[pl.ds(wid * nb, nb)], dst_v)
        pltpu.sync_copy(ones_hbm, ones_v)
        pltpu.sync_copy(zeros_hbm, acc.at[pl.ds(s * ROWS_PER_TILE, ROWS_PER_TILE)])
        plsc.subcore_barrier()

        @pl.loop(0, nb)
        def _(i):
            pltpu.sync_copy(ones_v, acc.at[dst_v.at[i]], add=True)

        plsc.subcore_barrier()
        pltpu.sync_copy(
            acc.at[pl.ds(s * ROWS_PER_TILE, ROWS_PER_TILE)],
            out_hbm.at[c, pl.ds(s * ROWS_PER_TILE, ROWS_PER_TILE)],
        )

    return deg_kernel


def _make_agg_kernel(e_pad, batch, nbuf, chb=16):
    nb = e_pad // (NW * batch)
    assert nb % chb == 0 and chb % nbuf == 0
    nchunk = nb // chb
    nin = chb // nbuf

    @functools.partial(
        pl.kernel,
        out_type=jax.ShapeDtypeStruct((NC, N_PAD, D), jnp.float32),
        mesh=_mesh,
        scratch_types=[
            pltpu.VMEM((chb, batch), jnp.int32),
            pltpu.VMEM((chb, batch), jnp.int32),
            pltpu.VMEM((nbuf, batch, D), jnp.float32),
            pltpu.VMEM_SHARED((N_PAD, D), jnp.float32),
            [pltpu.SemaphoreType.DMA] * nbuf,
            [pltpu.SemaphoreType.DMA] * nbuf,
        ],
    )
    def agg_kernel(g_hbm, src_hbm, dst_hbm, zeros_hbm, out_hbm,
                   src_v, dst_v, rows_v, acc, sems_g, sems_s):
        c = lax.axis_index("c")
        s = lax.axis_index("s")
        wid = c * NS + s
        pltpu.sync_copy(zeros_hbm, acc.at[pl.ds(s * ROWS_PER_TILE, ROWS_PER_TILE)])
        plsc.subcore_barrier()

        def gather(i, j):
            pltpu.async_copy(g_hbm.at[src_v.at[i]], rows_v.at[j], sems_g[j])

        def gather_wait(i, j):
            pltpu.make_async_copy(g_hbm.at[src_v.at[i]], rows_v.at[j],
                                  sems_g[j]).wait()

        def scatter(i, j):
            pltpu.async_copy(rows_v.at[j], acc.at[dst_v.at[i]], sems_s[j],
                             add=True)

        def scatter_wait(i, j):
            pltpu.make_async_copy(rows_v.at[j], acc.at[dst_v.at[i]],
                                  sems_s[j]).wait()

        @pl.loop(0, nchunk)
        def _(cidx):
            base = wid * nb + cidx * chb
            pltpu.sync_copy(src_hbm.at[pl.ds(base, chb)], src_v)
            pltpu.sync_copy(dst_hbm.at[pl.ds(base, chb)], dst_v)
            for j in range(nbuf):
                gather(j, j)

            @pl.loop(0, nin)
            def _(t):
                for j in range(nbuf):
                    k = t * nbuf + j
                    gather_wait(k, j)
                    scatter(k, j)
                for j in range(nbuf):
                    k = t * nbuf + j

                    @pl.when(t + 1 < nin)
                    def _():
                        scatter_wait(k, j)
                        gather(k + nbuf, j)

            for j in range(nbuf):
                scatter_wait(chb - nbuf + j, j)

        plsc.subcore_barrier()
        pltpu.sync_copy(
            acc.at[pl.ds(s * ROWS_PER_TILE, ROWS_PER_TILE)],
            out_hbm.at[c, pl.ds(s * ROWS_PER_TILE, ROWS_PER_TILE)],
        )

    return agg_kernel


_R1 = 1024


def _tc1_body(cnt0, cnt1, x, w, g_out):
    dinv = lax.rsqrt(cnt0[...] + cnt1[...] + 1.0)
    g_out[...] = dinv * jnp.dot(x[...], w[...], preferred_element_type=jnp.float32)


def _tc2_body(cnt0, cnt1, g1, p0, p1, b, w, g_out):
    dinv = lax.rsqrt(cnt0[...] + cnt1[...] + 1.0)
    h = dinv * (g1[...] + p0[...] + p1[...]) + b[...]
    g_out[...] = dinv * jnp.dot(h, w[...], preferred_element_type=jnp.float32)


def _tc3_body(cnt0, cnt1, g2, q0, q1, b, out):
    dinv = lax.rsqrt(cnt0[...] + cnt1[...] + 1.0)
    out[...] = dinv * (g2[...] + q0[...] + q1[...]) + b[...]


def _col_spec(r):
    return pl.BlockSpec((r, 1), lambda i: (i, 0))


def _mat_spec(r):
    return pl.BlockSpec((r, D), lambda i: (i, 0))


def _full_spec(shape):
    return pl.BlockSpec(shape, lambda i: tuple(0 for _ in shape))


def _tc1(cnt0, cnt1, x, w):
    return pl.pallas_call(
        _tc1_body,
        out_shape=jax.ShapeDtypeStruct((N_PAD, D), jnp.float32),
        grid=(N_PAD // _R1,),
        in_specs=[_col_spec(_R1), _col_spec(_R1), _mat_spec(_R1), _full_spec((D, D))],
        out_specs=_mat_spec(_R1),
    )(cnt0, cnt1, x, w)


def _tc2(cnt0, cnt1, g1, p0, p1, b, w):
    return pl.pallas_call(
        _tc2_body,
        out_shape=jax.ShapeDtypeStruct((N_PAD, D), jnp.float32),
        grid=(N_PAD // _R1,),
        in_specs=[_col_spec(_R1), _col_spec(_R1), _mat_spec(_R1), _mat_spec(_R1),
                  _mat_spec(_R1), _full_spec((1, D)), _full_spec((D, D))],
        out_specs=_mat_spec(_R1),
    )(cnt0, cnt1, g1, p0, p1, b, w)


def _tc3(cnt0, cnt1, g2, q0, q1, b):
    r = 1000
    return pl.pallas_call(
        _tc3_body,
        out_shape=jax.ShapeDtypeStruct((N, D), jnp.float32),
        grid=(N // r,),
        in_specs=[_col_spec(r), _col_spec(r), _mat_spec(r), _mat_spec(r),
                  _mat_spec(r), _full_spec((1, D))],
        out_specs=_mat_spec(r),
    )(cnt0, cnt1, g2, q0, q1, b)


def kernel(in_feat, adj, W1, b1, W2, b2):
    e = adj.shape[1]
    e_pad = _edge_pad(e)
    src = adj[0].astype(jnp.int32)
    dst = adj[1].astype(jnp.int32)
    pad = jnp.full((e_pad - e,), N, dtype=jnp.int32)
    src = jnp.concatenate([src, pad]).reshape(e_pad // BATCH, BATCH)
    dst = jnp.concatenate([dst, pad]).reshape(e_pad // BATCH, BATCH)

    x = jnp.zeros((N_PAD, D), jnp.float32).at[:N].set(in_feat)
    onesD = jnp.ones((BATCH, D), jnp.float32)
    zerosD = jnp.zeros((ROWS_PER_TILE, D), jnp.float32)

    deg_k = _make_deg_kernel(e_pad, D)
    agg_k = _make_agg_kernel(e_pad, BATCH, NBUF)

    cnt = deg_k(dst, onesD, zerosD)
    cnt0 = cnt[0, :, 0:1]
    cnt1 = cnt[1, :, 0:1]

    g1 = _tc1(cnt0, cnt1, x, W1)
    p = agg_k(g1, src, dst, zerosD)
    g2 = _tc2(cnt0, cnt1, g1, p[0], p[1], b1.reshape(1, D), W2)
    q = agg_k(g2, src, dst, zerosD)
    return _tc3(cnt0, cnt1, g2, q[0], q[1], b2.reshape(1, D))


# speedup vs baseline: 1.2031x; 1.2031x over previous
import functools

import jax
import jax.numpy as jnp
from jax import lax
from jax.experimental import pallas as pl
from jax.experimental.pallas import tpu as pltpu
from jax.experimental.pallas import tpu_sc as plsc

import math

N = 10000
D = 128
N_PAD = 10240
NC, NS = 2, 16
NW = NC * NS
BATCH = 64
NBUF = 4
ROWS_PER_TILE = N_PAD // NS

_mesh = plsc.VectorSubcoreMesh(core_axis_name="c", subcore_axis_name="s")


def _edge_pad(e):
    q = NW * BATCH * math.lcm(8, NBUF)
    return ((e + q - 1) // q) * q


def _make_deg_kernel(e_pad, w=16):
    nb = e_pad // (NW * BATCH)

    @functools.partial(
        pl.kernel,
        out_type=jax.ShapeDtypeStruct((NC, N_PAD, w), jnp.float32),
        mesh=_mesh,
        scratch_types=[
            pltpu.VMEM((nb, BATCH), jnp.int32),
            pltpu.VMEM((BATCH, w), jnp.float32),
            pltpu.VMEM_SHARED((N_PAD, w), jnp.float32),
        ],
    )
    def deg_kernel(dst_hbm, ones_hbm, zeros_hbm, out_hbm, dst_v, ones_v, acc):
        c = lax.axis_index("c")
        s = lax.axis_index("s")
        wid = c * NS + s
        pltpu.sync_copy(dst_hbm.at[pl.ds(wid * nb, nb)], dst_v)
        pltpu.sync_copy(ones_hbm, ones_v)
        pltpu.sync_copy(zeros_hbm, acc.at[pl.ds(s * ROWS_PER_TILE, ROWS_PER_TILE)])
        plsc.subcore_barrier()

        @pl.loop(0, nb)
        def _(i):
            pltpu.sync_copy(ones_v, acc.at[dst_v.at[i]], add=True)

        plsc.subcore_barrier()
        pltpu.sync_copy(
            acc.at[pl.ds(s * ROWS_PER_TILE, ROWS_PER_TILE)],
            out_hbm.at[c, pl.ds(s * ROWS_PER_TILE, ROWS_PER_TILE)],
        )

    return deg_kernel


def _make_agg_kernel(e_pad, batch, nbuf, chb=32):
    nb = e_pad // (NW * batch)
    assert nb % chb == 0 and chb % nbuf == 0
    nchunk = nb // chb
    nin = chb // nbuf

    @functools.partial(
        pl.kernel,
        out_type=jax.ShapeDtypeStruct((NC, N_PAD, D), jnp.float32),
        mesh=_mesh,
        scratch_types=[
            pltpu.VMEM((chb, batch), jnp.int32),
            pltpu.VMEM((chb, batch), jnp.int32),
            pltpu.VMEM((nbuf, batch, D), jnp.float32),
            pltpu.VMEM_SHARED((N_PAD, D), jnp.float32),
            [pltpu.SemaphoreType.DMA] * nbuf,
            [pltpu.SemaphoreType.DMA] * nbuf,
        ],
    )
    def agg_kernel(g_hbm, src_hbm, dst_hbm, zeros_hbm, out_hbm,
                   src_v, dst_v, rows_v, acc, sems_g, sems_s):
        c = lax.axis_index("c")
        s = lax.axis_index("s")
        wid = c * NS + s
        pltpu.sync_copy(zeros_hbm, acc.at[pl.ds(s * ROWS_PER_TILE, ROWS_PER_TILE)])
        plsc.subcore_barrier()

        def gather(i, j):
            pltpu.async_copy(g_hbm.at[src_v.at[i]], rows_v.at[j], sems_g[j])

        def gather_wait(i, j):
            pltpu.make_async_copy(g_hbm.at[src_v.at[i]], rows_v.at[j],
                                  sems_g[j]).wait()

        def scatter(i, j):
            pltpu.async_copy(rows_v.at[j], acc.at[dst_v.at[i]], sems_s[j],
                             add=True)

        def scatter_wait(i, j):
            pltpu.make_async_copy(rows_v.at[j], acc.at[dst_v.at[i]],
                                  sems_s[j]).wait()

        @pl.loop(0, nchunk)
        def _(cidx):
            base = wid * nb + cidx * chb
            pltpu.sync_copy(src_hbm.at[pl.ds(base, chb)], src_v)
            pltpu.sync_copy(dst_hbm.at[pl.ds(base, chb)], dst_v)
            for j in range(nbuf):
                gather(j, j)

            @pl.loop(0, nin)
            def _(t):
                for j in range(nbuf):
                    k = t * nbuf + j
                    gather_wait(k, j)
                    scatter(k, j)
                for j in range(nbuf):
                    k = t * nbuf + j

                    @pl.when(t + 1 < nin)
                    def _():
                        scatter_wait(k, j)
                        gather(k + nbuf, j)

            for j in range(nbuf):
                scatter_wait(chb - nbuf + j, j)

        plsc.subcore_barrier()
        pltpu.sync_copy(
            acc.at[pl.ds(s * ROWS_PER_TILE, ROWS_PER_TILE)],
            out_hbm.at[c, pl.ds(s * ROWS_PER_TILE, ROWS_PER_TILE)],
        )

    return agg_kernel


_R1 = 1024


def _tc1_body(cnt0, cnt1, x, w, g_out):
    dinv = lax.rsqrt(cnt0[...] + cnt1[...] + 1.0)
    g_out[...] = dinv * jnp.dot(x[...], w[...], preferred_element_type=jnp.float32)


def _tc2_body(cnt0, cnt1, g1, p0, p1, b, w, g_out):
    dinv = lax.rsqrt(cnt0[...] + cnt1[...] + 1.0)
    h = dinv * (g1[...] + p0[...] + p1[...]) + b[...]
    g_out[...] = dinv * jnp.dot(h, w[...], preferred_element_type=jnp.float32)


def _tc3_body(cnt0, cnt1, g2, q0, q1, b, out):
    dinv = lax.rsqrt(cnt0[...] + cnt1[...] + 1.0)
    out[...] = dinv * (g2[...] + q0[...] + q1[...]) + b[...]


def _col_spec(r):
    return pl.BlockSpec((r, 1), lambda i: (i, 0))


def _mat_spec(r):
    return pl.BlockSpec((r, D), lambda i: (i, 0))


def _full_spec(shape):
    return pl.BlockSpec(shape, lambda i: tuple(0 for _ in shape))


def _tc1(cnt0, cnt1, x, w):
    return pl.pallas_call(
        _tc1_body,
        out_shape=jax.ShapeDtypeStruct((N_PAD, D), jnp.float32),
        grid=(N_PAD // _R1,),
        in_specs=[_col_spec(_R1), _col_spec(_R1), _mat_spec(_R1), _full_spec((D, D))],
        out_specs=_mat_spec(_R1),
    )(cnt0, cnt1, x, w)


def _tc2(cnt0, cnt1, g1, p0, p1, b, w):
    return pl.pallas_call(
        _tc2_body,
        out_shape=jax.ShapeDtypeStruct((N_PAD, D), jnp.float32),
        grid=(N_PAD // _R1,),
        in_specs=[_col_spec(_R1), _col_spec(_R1), _mat_spec(_R1), _mat_spec(_R1),
                  _mat_spec(_R1), _full_spec((1, D)), _full_spec((D, D))],
        out_specs=_mat_spec(_R1),
    )(cnt0, cnt1, g1, p0, p1, b, w)


def _tc3(cnt0, cnt1, g2, q0, q1, b):
    r = 1000
    return pl.pallas_call(
        _tc3_body,
        out_shape=jax.ShapeDtypeStruct((N, D), jnp.float32),
        grid=(N // r,),
        in_specs=[_col_spec(r), _col_spec(r), _mat_spec(r), _mat_spec(r),
                  _mat_spec(r), _full_spec((1, D))],
        out_specs=_mat_spec(r),
    )(cnt0, cnt1, g2, q0, q1, b)


def kernel(in_feat, adj, W1, b1, W2, b2):
    e = adj.shape[1]
    e_pad = _edge_pad(e)
    src = adj[0].astype(jnp.int32)
    dst = adj[1].astype(jnp.int32)
    pad = jnp.full((e_pad - e,), N, dtype=jnp.int32)
    src = jnp.concatenate([src, pad]).reshape(e_pad // BATCH, BATCH)
    dst = jnp.concatenate([dst, pad]).reshape(e_pad // BATCH, BATCH)

    x = jnp.zeros((N_PAD, D), jnp.float32).at[:N].set(in_feat)
    onesD = jnp.ones((BATCH, D), jnp.float32)
    zerosD = jnp.zeros((ROWS_PER_TILE, D), jnp.float32)

    deg_k = _make_deg_kernel(e_pad, D)
    agg_k = _make_agg_kernel(e_pad, BATCH, NBUF)

    cnt = deg_k(dst, onesD, zerosD)
    cnt0 = cnt[0, :, 0:1]
    cnt1 = cnt[1, :, 0:1]

    g1 = _tc1(cnt0, cnt1, x, W1)
    p = agg_k(g1, src, dst, zerosD)
    g2 = _tc2(cnt0, cnt1, g1, p[0], p[1], b1.reshape(1, D), W2)
    q = agg_k(g2, src, dst, zerosD)
    return _tc3(cnt0, cnt1, g2, q[0], q[1], b2.reshape(1, D))


# speedup vs baseline: 1.2071x; 1.0034x over previous
import functools

import jax
import jax.numpy as jnp
from jax import lax
from jax.experimental import pallas as pl
from jax.experimental.pallas import tpu as pltpu
from jax.experimental.pallas import tpu_sc as plsc

import math

N = 10000
D = 128
N_PAD = 10240
NC, NS = 2, 16
NW = NC * NS
BATCH = 64
NBUF = 4
ROWS_PER_TILE = N_PAD // NS

_mesh = plsc.VectorSubcoreMesh(core_axis_name="c", subcore_axis_name="s")


def _edge_pad(e):
    q = NW * BATCH * math.lcm(8, NBUF)
    return ((e + q - 1) // q) * q


def _make_deg_kernel(e_pad, w=16):
    nb = e_pad // (NW * BATCH)

    @functools.partial(
        pl.kernel,
        out_type=jax.ShapeDtypeStruct((NC, N_PAD, w), jnp.float32),
        mesh=_mesh,
        scratch_types=[
            pltpu.VMEM((nb, BATCH), jnp.int32),
            pltpu.VMEM((BATCH, w), jnp.float32),
            pltpu.VMEM_SHARED((N_PAD, w), jnp.float32),
            [pltpu.SemaphoreType.DMA] * 4,
        ],
    )
    def deg_kernel(dst_hbm, ones_hbm, zeros_hbm, out_hbm, dst_v, ones_v, acc,
                   sems):
        c = lax.axis_index("c")
        s = lax.axis_index("s")
        wid = c * NS + s
        pltpu.sync_copy(dst_hbm.at[pl.ds(wid * nb, nb)], dst_v)
        pltpu.sync_copy(ones_hbm, ones_v)
        pltpu.sync_copy(zeros_hbm, acc.at[pl.ds(s * ROWS_PER_TILE, ROWS_PER_TILE)])
        plsc.subcore_barrier()

        def scatter(i, j):
            pltpu.async_copy(ones_v, acc.at[dst_v.at[i]], sems[j], add=True)

        def scatter_wait(i, j):
            pltpu.make_async_copy(ones_v, acc.at[dst_v.at[i]], sems[j]).wait()

        assert nb % 4 == 0
        for j in range(4):
            scatter(j, j)

        @pl.loop(0, nb // 4 - 1)
        def _(t):
            for j in range(4):
                i = t * 4 + j
                scatter_wait(i, j)
                scatter(i + 4, j)

        for j in range(4):
            scatter_wait(nb - 4 + j, j)

        plsc.subcore_barrier()
        pltpu.sync_copy(
            acc.at[pl.ds(s * ROWS_PER_TILE, ROWS_PER_TILE)],
            out_hbm.at[c, pl.ds(s * ROWS_PER_TILE, ROWS_PER_TILE)],
        )

    return deg_kernel


def _make_agg_kernel(e_pad, batch, nbuf, chb=32):
    nb = e_pad // (NW * batch)
    assert nb % chb == 0 and chb % nbuf == 0
    nchunk = nb // chb
    nin = chb // nbuf

    @functools.partial(
        pl.kernel,
        out_type=jax.ShapeDtypeStruct((NC, N_PAD, D), jnp.float32),
        mesh=_mesh,
        scratch_types=[
            pltpu.VMEM((chb, batch), jnp.int32),
            pltpu.VMEM((chb, batch), jnp.int32),
            pltpu.VMEM((nbuf, batch, D), jnp.float32),
            pltpu.VMEM_SHARED((N_PAD, D), jnp.float32),
            [pltpu.SemaphoreType.DMA] * nbuf,
            [pltpu.SemaphoreType.DMA] * nbuf,
        ],
    )
    def agg_kernel(g_hbm, src_hbm, dst_hbm, zeros_hbm, out_hbm,
                   src_v, dst_v, rows_v, acc, sems_g, sems_s):
        c = lax.axis_index("c")
        s = lax.axis_index("s")
        wid = c * NS + s
        pltpu.sync_copy(zeros_hbm, acc.at[pl.ds(s * ROWS_PER_TILE, ROWS_PER_TILE)])
        plsc.subcore_barrier()

        def gather(i, j):
            pltpu.async_copy(g_hbm.at[src_v.at[i]], rows_v.at[j], sems_g[j])

        def gather_wait(i, j):
            pltpu.make_async_copy(g_hbm.at[src_v.at[i]], rows_v.at[j],
                                  sems_g[j]).wait()

        def scatter(i, j):
            pltpu.async_copy(rows_v.at[j], acc.at[dst_v.at[i]], sems_s[j],
                             add=True)

        def scatter_wait(i, j):
            pltpu.make_async_copy(rows_v.at[j], acc.at[dst_v.at[i]],
                                  sems_s[j]).wait()

        @pl.loop(0, nchunk)
        def _(cidx):
            base = wid * nb + cidx * chb
            pltpu.sync_copy(src_hbm.at[pl.ds(base, chb)], src_v)
            pltpu.sync_copy(dst_hbm.at[pl.ds(base, chb)], dst_v)
            for j in range(nbuf):
                gather(j, j)

            @pl.loop(0, nin)
            def _(t):
                for j in range(nbuf):
                    k = t * nbuf + j
                    gather_wait(k, j)
                    scatter(k, j)
                for j in range(nbuf):
                    k = t * nbuf + j

                    @pl.when(t + 1 < nin)
                    def _():
                        scatter_wait(k, j)
                        gather(k + nbuf, j)

            for j in range(nbuf):
                scatter_wait(chb - nbuf + j, j)

        plsc.subcore_barrier()
        pltpu.sync_copy(
            acc.at[pl.ds(s * ROWS_PER_TILE, ROWS_PER_TILE)],
            out_hbm.at[c, pl.ds(s * ROWS_PER_TILE, ROWS_PER_TILE)],
        )

    return agg_kernel


_R1 = 1024


def _tc1_body(cnt0, cnt1, x, w, g_out):
    dinv = lax.rsqrt(cnt0[...] + cnt1[...] + 1.0)
    g_out[...] = dinv * jnp.dot(x[...], w[...], preferred_element_type=jnp.float32)


def _tc2_body(cnt0, cnt1, g1, p0, p1, b, w, g_out):
    dinv = lax.rsqrt(cnt0[...] + cnt1[...] + 1.0)
    h = dinv * (g1[...] + p0[...] + p1[...]) + b[...]
    g_out[...] = dinv * jnp.dot(h, w[...], preferred_element_type=jnp.float32)


def _tc3_body(cnt0, cnt1, g2, q0, q1, b, out):
    dinv = lax.rsqrt(cnt0[...] + cnt1[...] + 1.0)
    out[...] = dinv * (g2[...] + q0[...] + q1[...]) + b[...]


def _col_spec(r):
    return pl.BlockSpec((r, 1), lambda i: (i, 0))


def _mat_spec(r):
    return pl.BlockSpec((r, D), lambda i: (i, 0))


def _full_spec(shape):
    return pl.BlockSpec(shape, lambda i: tuple(0 for _ in shape))


def _tc1(cnt0, cnt1, x, w):
    return pl.pallas_call(
        _tc1_body,
        out_shape=jax.ShapeDtypeStruct((N_PAD, D), jnp.float32),
        grid=(N_PAD // _R1,),
        in_specs=[_col_spec(_R1), _col_spec(_R1), _mat_spec(_R1), _full_spec((D, D))],
        out_specs=_mat_spec(_R1),
    )(cnt0, cnt1, x, w)


def _tc2(cnt0, cnt1, g1, p0, p1, b, w):
    return pl.pallas_call(
        _tc2_body,
        out_shape=jax.ShapeDtypeStruct((N_PAD, D), jnp.float32),
        grid=(N_PAD // _R1,),
        in_specs=[_col_spec(_R1), _col_spec(_R1), _mat_spec(_R1), _mat_spec(_R1),
                  _mat_spec(_R1), _full_spec((1, D)), _full_spec((D, D))],
        out_specs=_mat_spec(_R1),
    )(cnt0, cnt1, g1, p0, p1, b, w)


def _tc3(cnt0, cnt1, g2, q0, q1, b):
    r = 1000
    return pl.pallas_call(
        _tc3_body,
        out_shape=jax.ShapeDtypeStruct((N, D), jnp.float32),
        grid=(N // r,),
        in_specs=[_col_spec(r), _col_spec(r), _mat_spec(r), _mat_spec(r),
                  _mat_spec(r), _full_spec((1, D))],
        out_specs=_mat_spec(r),
    )(cnt0, cnt1, g2, q0, q1, b)


def kernel(in_feat, adj, W1, b1, W2, b2):
    e = adj.shape[1]
    e_pad = _edge_pad(e)
    src = adj[0].astype(jnp.int32)
    dst = adj[1].astype(jnp.int32)
    pad = jnp.full((e_pad - e,), N, dtype=jnp.int32)
    src = jnp.concatenate([src, pad]).reshape(e_pad // BATCH, BATCH)
    dst = jnp.concatenate([dst, pad]).reshape(e_pad // BATCH, BATCH)

    x = jnp.zeros((N_PAD, D), jnp.float32).at[:N].set(in_feat)
    onesD = jnp.ones((BATCH, D), jnp.float32)
    zerosD = jnp.zeros((ROWS_PER_TILE, D), jnp.float32)

    deg_k = _make_deg_kernel(e_pad, D)
    agg_k = _make_agg_kernel(e_pad, BATCH, NBUF)

    cnt = deg_k(dst, onesD, zerosD)
    cnt0 = cnt[0, :, 0:1]
    cnt1 = cnt[1, :, 0:1]

    g1 = _tc1(cnt0, cnt1, x, W1)
    p = agg_k(g1, src, dst, zerosD)
    g2 = _tc2(cnt0, cnt1, g1, p[0], p[1], b1.reshape(1, D), W2)
    q = agg_k(g2, src, dst, zerosD)
    return _tc3(cnt0, cnt1, g2, q[0], q[1], b2.reshape(1, D))


# speedup vs baseline: 1.2142x; 1.0059x over previous
import functools

import jax
import jax.numpy as jnp
from jax import lax
from jax.experimental import pallas as pl
from jax.experimental.pallas import tpu as pltpu
from jax.experimental.pallas import tpu_sc as plsc

import math

N = 10000
D = 128
N_PAD = 10240
NC, NS = 2, 16
NW = NC * NS
BATCH = 64
NBUF = 4
ROWS_PER_TILE = N_PAD // NS

_mesh = plsc.VectorSubcoreMesh(core_axis_name="c", subcore_axis_name="s")


def _edge_pad(e):
    q = NW * BATCH * math.lcm(8, NBUF)
    return ((e + q - 1) // q) * q


def _make_deg_kernel(e_pad, w=16):
    nb = e_pad // (NW * BATCH)

    @functools.partial(
        pl.kernel,
        out_type=jax.ShapeDtypeStruct((NC, N_PAD, w), jnp.float32),
        mesh=_mesh,
        scratch_types=[
            pltpu.VMEM((nb, BATCH), jnp.int32),
            pltpu.VMEM((BATCH, w), jnp.float32),
            pltpu.VMEM_SHARED((N_PAD, w), jnp.float32),
            [pltpu.SemaphoreType.DMA] * 4,
        ],
    )
    def deg_kernel(dst_hbm, ones_hbm, zeros_hbm, out_hbm, dst_v, ones_v, acc,
                   sems):
        c = lax.axis_index("c")
        s = lax.axis_index("s")
        wid = c * NS + s
        pltpu.sync_copy(dst_hbm.at[pl.ds(wid * nb, nb)], dst_v)
        pltpu.sync_copy(ones_hbm, ones_v)
        pltpu.sync_copy(zeros_hbm, acc.at[pl.ds(s * ROWS_PER_TILE, ROWS_PER_TILE)])
        plsc.subcore_barrier()

        def scatter(i, j):
            pltpu.async_copy(ones_v, acc.at[dst_v.at[i]], sems[j], add=True)

        def scatter_wait(i, j):
            pltpu.make_async_copy(ones_v, acc.at[dst_v.at[i]], sems[j]).wait()

        assert nb % 4 == 0
        for j in range(4):
            scatter(j, j)

        @pl.loop(0, nb // 4 - 1)
        def _(t):
            for j in range(4):
                i = t * 4 + j
                scatter_wait(i, j)
                scatter(i + 4, j)

        for j in range(4):
            scatter_wait(nb - 4 + j, j)

        plsc.subcore_barrier()
        pltpu.sync_copy(
            acc.at[pl.ds(s * ROWS_PER_TILE, ROWS_PER_TILE)],
            out_hbm.at[c, pl.ds(s * ROWS_PER_TILE, ROWS_PER_TILE)],
        )

    return deg_kernel


def _make_agg_kernel(e_pad, batch, nbuf, chb=40):
    nb = e_pad // (NW * batch)
    assert nb % chb == 0 and chb % nbuf == 0
    nchunk = nb // chb
    nin = chb // nbuf

    @functools.partial(
        pl.kernel,
        out_type=jax.ShapeDtypeStruct((NC, N_PAD, D), jnp.float32),
        mesh=_mesh,
        scratch_types=[
            pltpu.VMEM((chb, batch), jnp.int32),
            pltpu.VMEM((chb, batch), jnp.int32),
            pltpu.VMEM((nbuf, batch, D), jnp.float32),
            pltpu.VMEM_SHARED((N_PAD, D), jnp.float32),
            [pltpu.SemaphoreType.DMA] * nbuf,
            [pltpu.SemaphoreType.DMA] * nbuf,
        ],
    )
    def agg_kernel(g_hbm, src_hbm, dst_hbm, zeros_hbm, out_hbm,
                   src_v, dst_v, rows_v, acc, sems_g, sems_s):
        c = lax.axis_index("c")
        s = lax.axis_index("s")
        wid = c * NS + s
        pltpu.sync_copy(zeros_hbm, acc.at[pl.ds(s * ROWS_PER_TILE, ROWS_PER_TILE)])
        plsc.subcore_barrier()

        def gather(i, j):
            pltpu.async_copy(g_hbm.at[src_v.at[i]], rows_v.at[j], sems_g[j])

        def gather_wait(i, j):
            pltpu.make_async_copy(g_hbm.at[src_v.at[i]], rows_v.at[j],
                                  sems_g[j]).wait()

        def scatter(i, j):
            pltpu.async_copy(rows_v.at[j], acc.at[dst_v.at[i]], sems_s[j],
                             add=True)

        def scatter_wait(i, j):
            pltpu.make_async_copy(rows_v.at[j], acc.at[dst_v.at[i]],
                                  sems_s[j]).wait()

        @pl.loop(0, nchunk)
        def _(cidx):
            base = wid * nb + cidx * chb
            pltpu.sync_copy(src_hbm.at[pl.ds(base, chb)], src_v)
            pltpu.sync_copy(dst_hbm.at[pl.ds(base, chb)], dst_v)
            for j in range(nbuf):
                gather(j, j)

            @pl.loop(0, nin)
            def _(t):
                for j in range(nbuf):
                    k = t * nbuf + j
                    gather_wait(k, j)
                    scatter(k, j)
                for j in range(nbuf):
                    k = t * nbuf + j

                    @pl.when(t + 1 < nin)
                    def _():
                        scatter_wait(k, j)
                        gather(k + nbuf, j)

            for j in range(nbuf):
                scatter_wait(chb - nbuf + j, j)

        plsc.subcore_barrier()
        pltpu.sync_copy(
            acc.at[pl.ds(s * ROWS_PER_TILE, ROWS_PER_TILE)],
            out_hbm.at[c, pl.ds(s * ROWS_PER_TILE, ROWS_PER_TILE)],
        )

    return agg_kernel


_R1 = 1024


def _tc1_body(cnt0, cnt1, x, w, g_out):
    dinv = lax.rsqrt(cnt0[...] + cnt1[...] + 1.0)
    g_out[...] = dinv * jnp.dot(x[...], w[...], preferred_element_type=jnp.float32)


def _tc2_body(cnt0, cnt1, g1, p0, p1, b, w, g_out):
    dinv = lax.rsqrt(cnt0[...] + cnt1[...] + 1.0)
    h = dinv * (g1[...] + p0[...] + p1[...]) + b[...]
    g_out[...] = dinv * jnp.dot(h, w[...], preferred_element_type=jnp.float32)


def _tc3_body(cnt0, cnt1, g2, q0, q1, b, out):
    dinv = lax.rsqrt(cnt0[...] + cnt1[...] + 1.0)
    out[...] = dinv * (g2[...] + q0[...] + q1[...]) + b[...]


def _col_spec(r):
    return pl.BlockSpec((r, 1), lambda i: (i, 0))


def _mat_spec(r):
    return pl.BlockSpec((r, D), lambda i: (i, 0))


def _full_spec(shape):
    return pl.BlockSpec(shape, lambda i: tuple(0 for _ in shape))


def _tc1(cnt0, cnt1, x, w):
    return pl.pallas_call(
        _tc1_body,
        out_shape=jax.ShapeDtypeStruct((N_PAD, D), jnp.float32),
        grid=(N_PAD // _R1,),
        in_specs=[_col_spec(_R1), _col_spec(_R1), _mat_spec(_R1), _full_spec((D, D))],
        out_specs=_mat_spec(_R1),
    )(cnt0, cnt1, x, w)


def _tc2(cnt0, cnt1, g1, p0, p1, b, w):
    return pl.pallas_call(
        _tc2_body,
        out_shape=jax.ShapeDtypeStruct((N_PAD, D), jnp.float32),
        grid=(N_PAD // _R1,),
        in_specs=[_col_spec(_R1), _col_spec(_R1), _mat_spec(_R1), _mat_spec(_R1),
                  _mat_spec(_R1), _full_spec((1, D)), _full_spec((D, D))],
        out_specs=_mat_spec(_R1),
    )(cnt0, cnt1, g1, p0, p1, b, w)


def _tc3(cnt0, cnt1, g2, q0, q1, b):
    r = 1000
    return pl.pallas_call(
        _tc3_body,
        out_shape=jax.ShapeDtypeStruct((N, D), jnp.float32),
        grid=(N // r,),
        in_specs=[_col_spec(r), _col_spec(r), _mat_spec(r), _mat_spec(r),
                  _mat_spec(r), _full_spec((1, D))],
        out_specs=_mat_spec(r),
    )(cnt0, cnt1, g2, q0, q1, b)


def kernel(in_feat, adj, W1, b1, W2, b2):
    e = adj.shape[1]
    e_pad = _edge_pad(e)
    src = adj[0].astype(jnp.int32)
    dst = adj[1].astype(jnp.int32)
    pad = jnp.full((e_pad - e,), N, dtype=jnp.int32)
    src = jnp.concatenate([src, pad]).reshape(e_pad // BATCH, BATCH)
    dst = jnp.concatenate([dst, pad]).reshape(e_pad // BATCH, BATCH)

    x = jnp.zeros((N_PAD, D), jnp.float32).at[:N].set(in_feat)
    onesD = jnp.ones((BATCH, D), jnp.float32)
    zerosD = jnp.zeros((ROWS_PER_TILE, D), jnp.float32)

    deg_k = _make_deg_kernel(e_pad, D)
    agg_k = _make_agg_kernel(e_pad, BATCH, NBUF)

    cnt = deg_k(dst, onesD, zerosD)
    cnt0 = cnt[0, :, 0:1]
    cnt1 = cnt[1, :, 0:1]

    g1 = _tc1(cnt0, cnt1, x, W1)
    p = agg_k(g1, src, dst, zerosD)
    g2 = _tc2(cnt0, cnt1, g1, p[0], p[1], b1.reshape(1, D), W2)
    q = agg_k(g2, src, dst, zerosD)
    return _tc3(cnt0, cnt1, g2, q[0], q[1], b2.reshape(1, D))


# speedup vs baseline: 1.2144x; 1.0001x over previous
import functools

import jax
import jax.numpy as jnp
from jax import lax
from jax.experimental import pallas as pl
from jax.experimental.pallas import tpu as pltpu
from jax.experimental.pallas import tpu_sc as plsc

import math

N = 10000
D = 128
N_PAD = 10240
NC, NS = 2, 16
NW = NC * NS
BATCH = 64
NBUF = 4
ROWS_PER_TILE = N_PAD // NS

_mesh = plsc.VectorSubcoreMesh(core_axis_name="c", subcore_axis_name="s")


def _edge_pad(e):
    q = NW * BATCH * math.lcm(8, NBUF)
    return ((e + q - 1) // q) * q


def _make_deg_kernel(e_pad, w=16):
    nb = e_pad // (NW * BATCH)

    @functools.partial(
        pl.kernel,
        out_type=jax.ShapeDtypeStruct((NC, N_PAD, w), jnp.float32),
        mesh=_mesh,
        scratch_types=[
            pltpu.VMEM((nb, BATCH), jnp.int32),
            pltpu.VMEM((BATCH, w), jnp.float32),
            pltpu.VMEM_SHARED((N_PAD, w), jnp.float32),
            [pltpu.SemaphoreType.DMA] * 4,
        ],
    )
    def deg_kernel(dst_hbm, ones_hbm, zeros_hbm, out_hbm, dst_v, ones_v, acc,
                   sems):
        c = lax.axis_index("c")
        s = lax.axis_index("s")
        wid = c * NS + s
        pltpu.sync_copy(dst_hbm.at[pl.ds(wid * nb, nb)], dst_v)
        pltpu.sync_copy(ones_hbm, ones_v)
        pltpu.sync_copy(zeros_hbm, acc.at[pl.ds(s * ROWS_PER_TILE, ROWS_PER_TILE)])
        plsc.subcore_barrier()

        def scatter(i, j):
            pltpu.async_copy(ones_v, acc.at[dst_v.at[i]], sems[j], add=True)

        def scatter_wait(i, j):
            pltpu.make_async_copy(ones_v, acc.at[dst_v.at[i]], sems[j]).wait()

        assert nb % 4 == 0
        for j in range(4):
            scatter(j, j)

        @pl.loop(0, nb // 4 - 1)
        def _(t):
            for j in range(4):
                i = t * 4 + j
                scatter_wait(i, j)
                scatter(i + 4, j)

        for j in range(4):
            scatter_wait(nb - 4 + j, j)

        plsc.subcore_barrier()
        pltpu.sync_copy(
            acc.at[pl.ds(s * ROWS_PER_TILE, ROWS_PER_TILE)],
            out_hbm.at[c, pl.ds(s * ROWS_PER_TILE, ROWS_PER_TILE)],
        )

    return deg_kernel


def _make_agg_kernel(e_pad, batch, nbuf, chb=40, split=0.5):
    tb = e_pad // (NS * batch)
    nb0 = int(round(tb * split / chb)) * chb
    nb1 = tb - nb0
    assert nb0 % chb == 0 and nb1 % chb == 0 and chb % nbuf == 0
    nin = chb // nbuf

    @functools.partial(
        pl.kernel,
        out_type=jax.ShapeDtypeStruct((NC, N_PAD, D), jnp.float32),
        mesh=_mesh,
        scratch_types=[
            pltpu.VMEM((chb, batch), jnp.int32),
            pltpu.VMEM((chb, batch), jnp.int32),
            pltpu.VMEM((nbuf, batch, D), jnp.float32),
            pltpu.VMEM_SHARED((N_PAD, D), jnp.float32),
            [pltpu.SemaphoreType.DMA] * nbuf,
            [pltpu.SemaphoreType.DMA] * nbuf,
        ],
    )
    def agg_kernel(g_hbm, src_hbm, dst_hbm, zeros_hbm, out_hbm,
                   src_v, dst_v, rows_v, acc, sems_g, sems_s):
        c = lax.axis_index("c")
        s = lax.axis_index("s")
        wid = c * NS + s
        pltpu.sync_copy(zeros_hbm, acc.at[pl.ds(s * ROWS_PER_TILE, ROWS_PER_TILE)])
        plsc.subcore_barrier()

        def gather(i, j):
            pltpu.async_copy(g_hbm.at[src_v.at[i]], rows_v.at[j], sems_g[j])

        def gather_wait(i, j):
            pltpu.make_async_copy(g_hbm.at[src_v.at[i]], rows_v.at[j],
                                  sems_g[j]).wait()

        def scatter(i, j):
            pltpu.async_copy(rows_v.at[j], acc.at[dst_v.at[i]], sems_s[j],
                             add=True)

        def scatter_wait(i, j):
            pltpu.make_async_copy(rows_v.at[j], acc.at[dst_v.at[i]],
                                  sems_s[j]).wait()

        def run(tile_base, nb_c):
            @pl.loop(0, nb_c // chb)
            def _(cidx):
                base = tile_base + cidx * chb
                pltpu.sync_copy(src_hbm.at[pl.ds(base, chb)], src_v)
                pltpu.sync_copy(dst_hbm.at[pl.ds(base, chb)], dst_v)
                for j in range(nbuf):
                    gather(j, j)

                @pl.loop(0, nin)
                def _(t):
                    for j in range(nbuf):
                        k = t * nbuf + j
                        gather_wait(k, j)
                        scatter(k, j)
                    for j in range(nbuf):
                        k = t * nbuf + j

                        @pl.when(t + 1 < nin)
                        def _():
                            scatter_wait(k, j)
                            gather(k + nbuf, j)

                for j in range(nbuf):
                    scatter_wait(chb - nbuf + j, j)

        if nb0 == nb1:
            run(wid * nb0, nb0)
        else:
            @pl.when(c == 0)
            def _():
                run(s * nb0, nb0)

            @pl.when(c == 1)
            def _():
                run(NS * nb0 + s * nb1, nb1)

        plsc.subcore_barrier()
        pltpu.sync_copy(
            acc.at[pl.ds(s * ROWS_PER_TILE, ROWS_PER_TILE)],
            out_hbm.at[c, pl.ds(s * ROWS_PER_TILE, ROWS_PER_TILE)],
        )

    return agg_kernel


_R1 = 1024


def _tc1_body(cnt0, cnt1, x, w, g_out):
    dinv = lax.rsqrt(cnt0[...] + cnt1[...] + 1.0)
    g_out[...] = dinv * jnp.dot(x[...], w[...], preferred_element_type=jnp.float32)


def _tc2_body(cnt0, cnt1, g1, p0, p1, b, w, g_out):
    dinv = lax.rsqrt(cnt0[...] + cnt1[...] + 1.0)
    h = dinv * (g1[...] + p0[...] + p1[...]) + b[...]
    g_out[...] = dinv * jnp.dot(h, w[...], preferred_element_type=jnp.float32)


def _tc3_body(cnt0, cnt1, g2, q0, q1, b, out):
    dinv = lax.rsqrt(cnt0[...] + cnt1[...] + 1.0)
    out[...] = dinv * (g2[...] + q0[...] + q1[...]) + b[...]


def _col_spec(r):
    return pl.BlockSpec((r, 1), lambda i: (i, 0))


def _mat_spec(r):
    return pl.BlockSpec((r, D), lambda i: (i, 0))


def _full_spec(shape):
    return pl.BlockSpec(shape, lambda i: tuple(0 for _ in shape))


def _tc1(cnt0, cnt1, x, w):
    return pl.pallas_call(
        _tc1_body,
        out_shape=jax.ShapeDtypeStruct((N_PAD, D), jnp.float32),
        grid=(N_PAD // _R1,),
        in_specs=[_col_spec(_R1), _col_spec(_R1), _mat_spec(_R1), _full_spec((D, D))],
        out_specs=_mat_spec(_R1),
    )(cnt0, cnt1, x, w)


def _tc2(cnt0, cnt1, g1, p0, p1, b, w):
    return pl.pallas_call(
        _tc2_body,
        out_shape=jax.ShapeDtypeStruct((N_PAD, D), jnp.float32),
        grid=(N_PAD // _R1,),
        in_specs=[_col_spec(_R1), _col_spec(_R1), _mat_spec(_R1), _mat_spec(_R1),
                  _mat_spec(_R1), _full_spec((1, D)), _full_spec((D, D))],
        out_specs=_mat_spec(_R1),
    )(cnt0, cnt1, g1, p0, p1, b, w)


def _tc3(cnt0, cnt1, g2, q0, q1, b):
    r = 1000
    return pl.pallas_call(
        _tc3_body,
        out_shape=jax.ShapeDtypeStruct((N, D), jnp.float32),
        grid=(N // r,),
        in_specs=[_col_spec(r), _col_spec(r), _mat_spec(r), _mat_spec(r),
                  _mat_spec(r), _full_spec((1, D))],
        out_specs=_mat_spec(r),
    )(cnt0, cnt1, g2, q0, q1, b)


def kernel(in_feat, adj, W1, b1, W2, b2):
    e = adj.shape[1]
    e_pad = _edge_pad(e)
    src = adj[0].astype(jnp.int32)
    dst = adj[1].astype(jnp.int32)
    pad = jnp.full((e_pad - e,), N, dtype=jnp.int32)
    src = jnp.concatenate([src, pad]).reshape(e_pad // BATCH, BATCH)
    dst = jnp.concatenate([dst, pad]).reshape(e_pad // BATCH, BATCH)

    x = jnp.zeros((N_PAD, D), jnp.float32).at[:N].set(in_feat)
    onesD = jnp.ones((BATCH, D), jnp.float32)
    zerosD = jnp.zeros((ROWS_PER_TILE, D), jnp.float32)

    deg_k = _make_deg_kernel(e_pad, D)
    agg_k = _make_agg_kernel(e_pad, BATCH, NBUF)

    cnt = deg_k(dst, onesD, zerosD)
    cnt0 = cnt[0, :, 0:1]
    cnt1 = cnt[1, :, 0:1]

    g1 = _tc1(cnt0, cnt1, x, W1)
    p = agg_k(g1, src, dst, zerosD)
    g2 = _tc2(cnt0, cnt1, g1, p[0], p[1], b1.reshape(1, D), W2)
    q = agg_k(g2, src, dst, zerosD)
    return _tc3(cnt0, cnt1, g2, q[0], q[1], b2.reshape(1, D))
